# baseline (speedup 1.0000x reference)
import jax
import jax.numpy as jnp
from jax import lax
from jax.experimental import pallas as pl
from jax.experimental.pallas import tpu as pltpu
from jax.experimental.pallas import tpu_sc as plsc

_B, _S, _L, _V, _D = 1024, 5, 20, 1024, 64
_MAX_NORM = 20.0
_K_NEG = 3
_LP = 24
_NC, _NS = 2, 16
_NW = _NC * _NS
_DPW = _B // _NW
_EXT = _K_NEG
_NDOC = _DPW + _EXT
_W = 2 * _L
_RPT = _V // _NS
_DOC_CH = 3
_CHUNKS = [(c * _DOC_CH, min(_DOC_CH, _NDOC - c * _DOC_CH))
           for c in range((_NDOC + _DOC_CH - 1) // _DOC_CH)]
_GROUPS = 3
_NCMP = (_NDOC * _W + 15) // 16


def _sc_body(docs_hbm, tbl_hbm, l_hbm, r_hbm, neg_hbm,
             dv_v, ab_v, tt_v, trow_v, scales_v, rows_v, lt_v, rt_v, nt_v,
             tn_sh, sem_i, sem_t, sem_o, *sem_g):
    cid = lax.axis_index("c")
    sid = lax.axis_index("s")
    wid = sid * _NC + cid
    base = wid * _DPW
    b2 = lax.rem(base + _DPW, _B)

    st_t = pltpu.async_copy(tbl_hbm.at[:, pl.ds(sid * _RPT, _RPT)], tt_v,
                            sem_t)
    st_d0 = pltpu.async_copy(docs_hbm.at[:, :, pl.ds(base, _DPW)],
                             dv_v.at[:, :, pl.ds(0, _DPW)], sem_i)
    st_d1 = pltpu.async_copy(docs_hbm.at[:, :, pl.ds(b2, 8)],
                             dv_v.at[:, :, pl.ds(_DPW, 8)], sem_i)

    with jax.named_scope("wait_t"):
        st_t.wait()

    for vc in range(4):
        def ssbody(d, acc):
            x = tt_v[d, pl.ds(vc * 16, 16)]
            return acc + x * x

        ssv = lax.fori_loop(0, _D, ssbody, jnp.zeros((16,), jnp.float32))
        yi = plsc.bitcast(ssv, jnp.int32)
        yi = 0x5F3759DF - lax.shift_right_logical(yi, 1)
        y = plsc.bitcast(yi, jnp.float32)
        h = 0.5 * ssv
        for _ in range(3):
            y = y * (1.5 - h * y * y)
        scales_v[pl.ds(vc * 16, 16)] = jnp.minimum(1.0, _MAX_NORM * y)

    def scbody(d, carry):
        dvec = jnp.full((16,), d, jnp.int32)
        for vc in range(4):
            vvec = lax.iota(jnp.int32, 16) + vc * 16
            x = tt_v[d, pl.ds(vc * 16, 16)] * scales_v[pl.ds(vc * 16, 16)]
            plsc.store_scatter(trow_v, [vvec, dvec], x)
        return carry

    lax.fori_loop(0, _D, scbody, 0)
    with jax.named_scope("publish"):
        pltpu.sync_copy(trow_v, tn_sh.at[pl.ds(sid * _RPT, _RPT)])
        plsc.subcore_barrier()

    with jax.named_scope("wait_docs"):
        st_d0.wait()
        st_d1.wait()

    def cbody(g, carry):
        pv = lax.iota(jnp.int32, 16) + g * 16
        m = jnp.minimum(pv // _W, _NDOC - 1)
        off = pv - (pv // _W) * _W
        s = off // _L
        t = off - s * _L
        ab_v[pl.ds(g * 16, 16)] = plsc.load_gather(dv_v, [s, t, m])
        return carry

    with jax.named_scope("compact"):
        lax.fori_loop(0, _NCMP, cbody, 0)

    cps = []
    for ci, (doc0, nd) in enumerate(_CHUNKS):
        cps.append(pltpu.async_copy(
            tn_sh.at[ab_v.at[pl.ds(doc0 * _W, nd * _W)]],
            rows_v.at[pl.ds(doc0 * _W, nd * _W)], sem_g[ci]))

    def doc_body(j, carry):
        rb = j * _W
        jvec = jnp.full((16,), j, jnp.int32)
        acc = [rows_v[rb, pl.ds(d * 16, 16)] for d in range(4)]
        for t in range(1, _L):
            for d in range(4):
                acc[d] = acc[d] + rows_v[rb + t, pl.ds(d * 16, 16)]

        @pl.when(j < _DPW)
        def _():
            for dc in range(4):
                dvec = lax.iota(jnp.int32, 16) + dc * 16
                plsc.store_scatter(lt_v, [dvec, jvec], acc[dc])

        for k in range(_K_NEG):
            m = j - 1 - k

            @pl.when(jnp.logical_and(m >= 0, m < _DPW))
            def _():
                kvec = jnp.full((16,), k, jnp.int32)
                mvec = jnp.full((16,), m, jnp.int32)
                for dc in range(4):
                    dvec = lax.iota(jnp.int32, 16) + dc * 16
                    plsc.store_scatter(nt_v, [kvec, dvec, mvec], acc[dc])

        @pl.when(j < _DPW)
        def _():
            accb = [rows_v[rb + _L, pl.ds(d * 16, 16)] for d in range(4)]
            for t in range(1, _L):
                for d in range(4):
                    accb[d] = accb[d] + rows_v[rb + _L + t, pl.ds(d * 16, 16)]
            for dc in range(4):
                dvec = lax.iota(jnp.int32, 16) + dc * 16
                plsc.store_scatter(rt_v, [dvec, jvec], accb[dc])
        return carry

    ng = (len(_CHUNKS) + _GROUPS - 1) // _GROUPS
    done = 0
    for gi, g in enumerate(range(0, len(_CHUNKS), ng)):
        grp = _CHUNKS[g:g + ng]
        with jax.named_scope(f"gwait{gi}"):
            for ci in range(g, g + len(grp)):
                cps[ci].wait()
        lo, hi = done, grp[-1][0] + grp[-1][1]
        with jax.named_scope(f"pool{gi}"):
            lax.fori_loop(lo, hi, doc_body, 0)
        done = hi

    out_l = pltpu.async_copy(lt_v, l_hbm.at[:, pl.ds(base, _DPW)], sem_o)
    out_r = pltpu.async_copy(rt_v, r_hbm.at[:, pl.ds(base, _DPW)], sem_o)
    out_n = pltpu.async_copy(nt_v, neg_hbm.at[:, :, pl.ds(base, _DPW)],
                             sem_o)
    with jax.named_scope("out"):
        out_l.wait()
        out_r.wait()
        out_n.wait()


_sc_embed = pl.kernel(
    _sc_body,
    out_type=(
        jax.ShapeDtypeStruct((_D, _B), jnp.float32),
        jax.ShapeDtypeStruct((_D, _B), jnp.float32),
        jax.ShapeDtypeStruct((_K_NEG, _D, _B), jnp.float32),
    ),
    mesh=plsc.VectorSubcoreMesh(core_axis_name="c", subcore_axis_name="s",
                                num_cores=_NC, num_subcores=_NS),
    scratch_types=[
        pltpu.VMEM((2, _LP, _DPW + 8), jnp.int32),
        pltpu.VMEM((_NCMP * 16,), jnp.int32),
        pltpu.VMEM((_D, _RPT), jnp.float32),
        pltpu.VMEM((_RPT, _D), jnp.float32),
        pltpu.VMEM((_RPT,), jnp.float32),
        pltpu.VMEM((_NDOC * _W, _D), jnp.float32),
        pltpu.VMEM((_D, _DPW), jnp.float32),
        pltpu.VMEM((_D, _DPW), jnp.float32),
        pltpu.VMEM((_K_NEG, _D, _DPW), jnp.float32),
        pltpu.VMEM_SHARED((_V, _D), jnp.float32),
        pltpu.SemaphoreType.DMA,
        pltpu.SemaphoreType.DMA,
        pltpu.SemaphoreType.DMA,
    ] + [pltpu.SemaphoreType.DMA] * len(_CHUNKS),
    compiler_params=pltpu.CompilerParams(use_tc_tiling_on_sc=False,
                                         needs_layout_passes=False),
)


def kernel(docs, table):
    docs_t = jnp.pad(docs.transpose(1, 2, 0)[:2], ((0, 0), (0, _LP - _L),
                                                   (0, 0)))
    lt, rt, nt = _sc_embed(docs_t, table.T)
    l = lt.T[:, None, :]
    r = rt.T[:, None, :]
    neg = nt.transpose(2, 0, 1)
    return l, r, neg


# speedup vs baseline: 1.1420x; 1.1420x over previous
import jax
import jax.numpy as jnp
from jax import lax
from jax.experimental import pallas as pl
from jax.experimental.pallas import tpu as pltpu
from jax.experimental.pallas import tpu_sc as plsc

_B, _S, _L, _V, _D = 1024, 5, 20, 1024, 64
_MAX_NORM = 20.0
_K_NEG = 3
_LP = 24
_NC, _NS = 2, 16
_NW = _NC * _NS
_DPW = _B // _NW
_EXT = _K_NEG
_NDOC = _DPW + _EXT
_W = 2 * _L
_RPT = _V // _NS
_DOC_CH = 3
_CHUNKS = [(c * _DOC_CH, min(_DOC_CH, _NDOC - c * _DOC_CH))
           for c in range((_NDOC + _DOC_CH - 1) // _DOC_CH)]
_GROUPS = 3
_NCMP = (_NDOC * _W + 15) // 16


def _sc_body(docs_hbm, tbl_hbm, l_hbm, r_hbm, neg_hbm,
             dv_v, ab_v, tt_v, trow_v, scales_v, rows_v, lt_v, rt_v, nt_v,
             tn_sh, sem_i, sem_t, sem_o, *sem_g):
    cid = lax.axis_index("c")
    sid = lax.axis_index("s")
    wid = sid * _NC + cid
    base = wid * _DPW
    b2 = lax.rem(base + _DPW, _B)

    st_t = pltpu.async_copy(tbl_hbm.at[:, pl.ds(sid * _RPT, _RPT)], tt_v,
                            sem_t)
    st_d0 = pltpu.async_copy(docs_hbm.at[:, :, pl.ds(base, _DPW)],
                             dv_v.at[:, :, pl.ds(0, _DPW)], sem_i)
    st_d1 = pltpu.async_copy(docs_hbm.at[:, :, pl.ds(b2, 8)],
                             dv_v.at[:, :, pl.ds(_DPW, 8)], sem_i)

    with jax.named_scope("wait_t"):
        st_t.wait()

    def ssbody(d, accs):
        xs = [tt_v[d, pl.ds(vc * 16, 16)] for vc in range(4)]
        return tuple(accs[vc] + xs[vc] * xs[vc] for vc in range(4))

    sss = lax.fori_loop(0, _D, ssbody,
                        tuple(jnp.zeros((16,), jnp.float32)
                              for _ in range(4)))
    for vc in range(4):
        ssv = sss[vc]
        yi = plsc.bitcast(ssv, jnp.int32)
        yi = 0x5F3759DF - lax.shift_right_logical(yi, 1)
        y = plsc.bitcast(yi, jnp.float32)
        h = 0.5 * ssv
        for _ in range(3):
            y = y * (1.5 - h * y * y)
        scales_v[pl.ds(vc * 16, 16)] = jnp.minimum(1.0, _MAX_NORM * y)

    def scbody(d, carry):
        dvec = jnp.full((16,), d, jnp.int32)
        for vc in range(4):
            vvec = lax.iota(jnp.int32, 16) + vc * 16
            x = tt_v[d, pl.ds(vc * 16, 16)] * scales_v[pl.ds(vc * 16, 16)]
            plsc.store_scatter(trow_v, [vvec, dvec], x)
        return carry

    lax.fori_loop(0, _D, scbody, 0)
    with jax.named_scope("publish"):
        pltpu.sync_copy(trow_v.at[:, pl.ds(0, _D)],
                        tn_sh.at[pl.ds(sid * _RPT, _RPT)])
        plsc.subcore_barrier()

    with jax.named_scope("wait_docs"):
        st_d0.wait()
        st_d1.wait()

    def cbody(g, carry):
        pv = lax.iota(jnp.int32, 16) + g * 16
        m = jnp.minimum(pv // _W, _NDOC - 1)
        off = pv - (pv // _W) * _W
        s = off // _L
        t = off - s * _L
        ab_v[pl.ds(g * 16, 16)] = plsc.load_gather(dv_v, [s, t, m])
        return carry

    with jax.named_scope("compact"):
        lax.fori_loop(0, _NCMP, cbody, 0)

    cps = []
    for ci, (doc0, nd) in enumerate(_CHUNKS):
        cps.append(pltpu.async_copy(
            tn_sh.at[ab_v.at[pl.ds(doc0 * _W, nd * _W)]],
            rows_v.at[pl.ds(doc0 * _W, nd * _W)], sem_g[ci]))

    def doc_body(j, carry):
        rb = j * _W
        jvec = jnp.full((16,), j, jnp.int32)
        acc = [rows_v[rb, pl.ds(d * 16, 16)] for d in range(4)]
        for t in range(1, _L):
            for d in range(4):
                acc[d] = acc[d] + rows_v[rb + t, pl.ds(d * 16, 16)]

        @pl.when(j < _DPW)
        def _():
            for dc in range(4):
                dvec = lax.iota(jnp.int32, 16) + dc * 16
                plsc.store_scatter(lt_v, [dvec, jvec], acc[dc])

        for k in range(_K_NEG):
            m = j - 1 - k

            @pl.when(jnp.logical_and(m >= 0, m < _DPW))
            def _():
                kvec = jnp.full((16,), k, jnp.int32)
                mvec = jnp.full((16,), m, jnp.int32)
                for dc in range(4):
                    dvec = lax.iota(jnp.int32, 16) + dc * 16
                    plsc.store_scatter(nt_v, [kvec, dvec, mvec], acc[dc])

        @pl.when(j < _DPW)
        def _():
            accb = [rows_v[rb + _L, pl.ds(d * 16, 16)] for d in range(4)]
            for t in range(1, _L):
                for d in range(4):
                    accb[d] = accb[d] + rows_v[rb + _L + t, pl.ds(d * 16, 16)]
            for dc in range(4):
                dvec = lax.iota(jnp.int32, 16) + dc * 16
                plsc.store_scatter(rt_v, [dvec, jvec], accb[dc])
        return carry

    ng = (len(_CHUNKS) + _GROUPS - 1) // _GROUPS
    done = 0
    for gi, g in enumerate(range(0, len(_CHUNKS), ng)):
        grp = _CHUNKS[g:g + ng]
        with jax.named_scope(f"gwait{gi}"):
            for ci in range(g, g + len(grp)):
                cps[ci].wait()
        lo, hi = done, grp[-1][0] + grp[-1][1]
        with jax.named_scope(f"pool{gi}"):
            lax.fori_loop(lo, hi, doc_body, 0)
        done = hi

    out_l = pltpu.async_copy(lt_v.at[:, pl.ds(0, _DPW)],
                             l_hbm.at[:, pl.ds(base, _DPW)], sem_o)
    out_r = pltpu.async_copy(rt_v.at[:, pl.ds(0, _DPW)],
                             r_hbm.at[:, pl.ds(base, _DPW)], sem_o)
    out_n = pltpu.async_copy(nt_v.at[:, :, pl.ds(0, _DPW)],
                             neg_hbm.at[:, :, pl.ds(base, _DPW)], sem_o)
    with jax.named_scope("out"):
        out_l.wait()
        out_r.wait()
        out_n.wait()


_sc_embed = pl.kernel(
    _sc_body,
    out_type=(
        jax.ShapeDtypeStruct((_D, _B), jnp.float32),
        jax.ShapeDtypeStruct((_D, _B), jnp.float32),
        jax.ShapeDtypeStruct((_K_NEG, _D, _B), jnp.float32),
    ),
    mesh=plsc.VectorSubcoreMesh(core_axis_name="c", subcore_axis_name="s",
                                num_cores=_NC, num_subcores=_NS),
    scratch_types=[
        pltpu.VMEM((2, _LP, _DPW + 9), jnp.int32),
        pltpu.VMEM((_NCMP * 16,), jnp.int32),
        pltpu.VMEM((_D, _RPT), jnp.float32),
        pltpu.VMEM((_RPT, _D + 1), jnp.float32),
        pltpu.VMEM((_RPT,), jnp.float32),
        pltpu.VMEM((_NDOC * _W, _D), jnp.float32),
        pltpu.VMEM((_D, _DPW + 1), jnp.float32),
        pltpu.VMEM((_D, _DPW + 1), jnp.float32),
        pltpu.VMEM((_K_NEG, _D, _DPW + 1), jnp.float32),
        pltpu.VMEM_SHARED((_V, _D), jnp.float32),
        pltpu.SemaphoreType.DMA,
        pltpu.SemaphoreType.DMA,
        pltpu.SemaphoreType.DMA,
    ] + [pltpu.SemaphoreType.DMA] * len(_CHUNKS),
    compiler_params=pltpu.CompilerParams(use_tc_tiling_on_sc=False,
                                         needs_layout_passes=False),
)


def kernel(docs, table):
    docs_t = jnp.pad(docs.transpose(1, 2, 0)[:2], ((0, 0), (0, _LP - _L),
                                                   (0, 0)))
    lt, rt, nt = _sc_embed(docs_t, table.T)
    l = lt.T[:, None, :]
    r = rt.T[:, None, :]
    neg = nt.transpose(2, 0, 1)
    return l, r, neg


# speedup vs baseline: 1.1755x; 1.0293x over previous
import jax
import jax.numpy as jnp
from jax import lax
from jax.experimental import pallas as pl
from jax.experimental.pallas import tpu as pltpu
from jax.experimental.pallas import tpu_sc as plsc

_B, _S, _L, _V, _D = 1024, 5, 20, 1024, 64
_MAX_NORM = 20.0
_K_NEG = 3
_LP = 24
_NC, _NS = 2, 16
_NW = _NC * _NS
_DPW = _B // _NW
_EXT = _K_NEG
_NDOC = _DPW + _EXT
_W = 2 * _L
_RPT = _V // _NS
_DOC_CH = 3
_CHUNKS = [(c * _DOC_CH, min(_DOC_CH, _NDOC - c * _DOC_CH))
           for c in range((_NDOC + _DOC_CH - 1) // _DOC_CH)]
_GROUPS = 3
_NCMP = (_NDOC * _W + 15) // 16


def _sc_body(docs_hbm, tbl_hbm, l_hbm, r_hbm, neg_hbm,
             dv_v, ab_v, tt_v, trow_v, scales_v, rows_v, lt_v, rt_v, nt_v,
             tn_sh, sem_i, sem_t, sem_o, *sem_g):
    cid = lax.axis_index("c")
    sid = lax.axis_index("s")
    wid = sid * _NC + cid
    base = wid * _DPW
    b2 = lax.rem(base + _DPW, _B)

    st_t = pltpu.async_copy(tbl_hbm.at[:, pl.ds(sid * _RPT, _RPT)], tt_v,
                            sem_t)
    st_d0 = pltpu.async_copy(docs_hbm.at[:, :, pl.ds(base, _DPW)],
                             dv_v.at[:, :, pl.ds(0, _DPW)], sem_i)
    st_d1 = pltpu.async_copy(docs_hbm.at[:, :, pl.ds(b2, 8)],
                             dv_v.at[:, :, pl.ds(_DPW, 8)], sem_i)

    with jax.named_scope("wait_docs"):
        st_d0.wait()
        st_d1.wait()

    with jax.named_scope("compact"):
        iota = lax.iota(jnp.int32, 16)
        for g in range(_NCMP):
            w0 = 16 * g
            m0, r0 = divmod(w0, _W)
            offv = iota + r0
            wrap = (offv >= _W).astype(jnp.int32)
            mv = jnp.minimum(m0 + wrap, _NDOC - 1)
            off2 = offv - _W * wrap
            sv = (off2 >= _L).astype(jnp.int32)
            tv = off2 - _L * sv
            ab_v[pl.ds(w0, 16)] = plsc.load_gather(dv_v, [sv, tv, mv])

    with jax.named_scope("wait_t"):
        st_t.wait()

    def ssbody(d, accs):
        xs = [tt_v[d, pl.ds(vc * 16, 16)] for vc in range(4)]
        return tuple(accs[vc] + xs[vc] * xs[vc] for vc in range(4))

    with jax.named_scope("ss"):
        sss = lax.fori_loop(0, _D, ssbody,
                            tuple(jnp.zeros((16,), jnp.float32)
                                  for _ in range(4)))
    for vc in range(4):
        ssv = sss[vc]
        yi = plsc.bitcast(ssv, jnp.int32)
        yi = 0x5F3759DF - lax.shift_right_logical(yi, 1)
        y = plsc.bitcast(yi, jnp.float32)
        h = 0.5 * ssv
        for _ in range(3):
            y = y * (1.5 - h * y * y)
        scales_v[pl.ds(vc * 16, 16)] = jnp.minimum(1.0, _MAX_NORM * y)

    def scbody(d, carry):
        dvec = jnp.full((16,), d, jnp.int32)
        for vc in range(4):
            vvec = lax.iota(jnp.int32, 16) + vc * 16
            x = tt_v[d, pl.ds(vc * 16, 16)] * scales_v[pl.ds(vc * 16, 16)]
            plsc.store_scatter(trow_v, [vvec, dvec], x)
        return carry

    with jax.named_scope("tscale"):
        lax.fori_loop(0, _D, scbody, 0)
    with jax.named_scope("publish"):
        pltpu.sync_copy(trow_v.at[:, pl.ds(0, _D)],
                        tn_sh.at[pl.ds(sid * _RPT, _RPT)])
        plsc.subcore_barrier()

    cps = []
    for ci, (doc0, nd) in enumerate(_CHUNKS):
        cps.append(pltpu.async_copy(
            tn_sh.at[ab_v.at[pl.ds(doc0 * _W, nd * _W)]],
            rows_v.at[pl.ds(doc0 * _W, nd * _W)], sem_g[ci]))

    def doc_body(j, carry):
        rb = j * _W
        jvec = jnp.full((16,), j, jnp.int32)
        acc = [rows_v[rb, pl.ds(d * 16, 16)] for d in range(4)]
        for t in range(1, _L):
            for d in range(4):
                acc[d] = acc[d] + rows_v[rb + t, pl.ds(d * 16, 16)]

        @pl.when(j < _DPW)
        def _():
            for dc in range(4):
                dvec = lax.iota(jnp.int32, 16) + dc * 16
                plsc.store_scatter(lt_v, [dvec, jvec], acc[dc])

        for k in range(_K_NEG):
            m = j - 1 - k

            @pl.when(jnp.logical_and(m >= 0, m < _DPW))
            def _():
                kvec = jnp.full((16,), k, jnp.int32)
                mvec = jnp.full((16,), m, jnp.int32)
                for dc in range(4):
                    dvec = lax.iota(jnp.int32, 16) + dc * 16
                    plsc.store_scatter(nt_v, [kvec, dvec, mvec], acc[dc])

        @pl.when(j < _DPW)
        def _():
            accb = [rows_v[rb + _L, pl.ds(d * 16, 16)] for d in range(4)]
            for t in range(1, _L):
                for d in range(4):
                    accb[d] = accb[d] + rows_v[rb + _L + t, pl.ds(d * 16, 16)]
            for dc in range(4):
                dvec = lax.iota(jnp.int32, 16) + dc * 16
                plsc.store_scatter(rt_v, [dvec, jvec], accb[dc])
        return carry

    ng = (len(_CHUNKS) + _GROUPS - 1) // _GROUPS
    done = 0
    for gi, g in enumerate(range(0, len(_CHUNKS), ng)):
        grp = _CHUNKS[g:g + ng]
        with jax.named_scope(f"gwait{gi}"):
            for ci in range(g, g + len(grp)):
                cps[ci].wait()
        lo, hi = done, grp[-1][0] + grp[-1][1]
        with jax.named_scope(f"pool{gi}"):
            lax.fori_loop(lo, hi, doc_body, 0)
        done = hi

    out_l = pltpu.async_copy(lt_v.at[:, pl.ds(0, _DPW)],
                             l_hbm.at[:, pl.ds(base, _DPW)], sem_o)
    out_r = pltpu.async_copy(rt_v.at[:, pl.ds(0, _DPW)],
                             r_hbm.at[:, pl.ds(base, _DPW)], sem_o)
    out_n = pltpu.async_copy(nt_v.at[:, :, pl.ds(0, _DPW)],
                             neg_hbm.at[:, :, pl.ds(base, _DPW)], sem_o)
    with jax.named_scope("out"):
        out_l.wait()
        out_r.wait()
        out_n.wait()


_sc_embed = pl.kernel(
    _sc_body,
    out_type=(
        jax.ShapeDtypeStruct((_D, _B), jnp.float32),
        jax.ShapeDtypeStruct((_D, _B), jnp.float32),
        jax.ShapeDtypeStruct((_K_NEG, _D, _B), jnp.float32),
    ),
    mesh=plsc.VectorSubcoreMesh(core_axis_name="c", subcore_axis_name="s",
                                num_cores=_NC, num_subcores=_NS),
    scratch_types=[
        pltpu.VMEM((2, _LP, _DPW + 9), jnp.int32),
        pltpu.VMEM((_NCMP * 16,), jnp.int32),
        pltpu.VMEM((_D, _RPT), jnp.float32),
        pltpu.VMEM((_RPT, _D + 1), jnp.float32),
        pltpu.VMEM((_RPT,), jnp.float32),
        pltpu.VMEM((_NDOC * _W, _D), jnp.float32),
        pltpu.VMEM((_D, _DPW + 1), jnp.float32),
        pltpu.VMEM((_D, _DPW + 1), jnp.float32),
        pltpu.VMEM((_K_NEG, _D, _DPW + 1), jnp.float32),
        pltpu.VMEM_SHARED((_V, _D), jnp.float32),
        pltpu.SemaphoreType.DMA,
        pltpu.SemaphoreType.DMA,
        pltpu.SemaphoreType.DMA,
    ] + [pltpu.SemaphoreType.DMA] * len(_CHUNKS),
    compiler_params=pltpu.CompilerParams(use_tc_tiling_on_sc=False,
                                         needs_layout_passes=False),
)


def kernel(docs, table):
    docs_t = jnp.pad(docs.transpose(1, 2, 0)[:2], ((0, 0), (0, _LP - _L),
                                                   (0, 0)))
    lt, rt, nt = _sc_embed(docs_t, table.T)
    l = lt.T[:, None, :]
    r = rt.T[:, None, :]
    neg = nt.transpose(2, 0, 1)
    return l, r, neg
